# baseline (speedup 1.0000x reference)
import functools

import jax
import jax.numpy as jnp
from jax import lax
from jax.experimental import pallas as pl
from jax.experimental.pallas import tpu as pltpu
from jax.experimental.pallas import tpu_sc as plsc

_M = 100000
_MPAD = 100352
_D = 128
_NL = 4
_B = 16384
_NW = 32
_BPW = _B // _NW
_SKCH = 2048
_RCH = 64
_NCH = _BPW // _RCH
_TRASH = 2048

_mesh = plsc.VectorSubcoreMesh(core_axis_name="c", subcore_axis_name="s")


@functools.partial(
    pl.kernel,
    mesh=_mesh,
    compiler_params=pltpu.CompilerParams(needs_layout_passes=False),
    out_type=[
        jax.ShapeDtypeStruct((_B + _TRASH, _D), jnp.float32),
        jax.ShapeDtypeStruct((_B * _NL,), jnp.float32),
    ],
    scratch_types=[
        pltpu.VMEM((_MPAD,), jnp.int32),
        pltpu.VMEM((_BPW,), jnp.int32),
        pltpu.VMEM((_SKCH,), jnp.int32),
        pltpu.VMEM((_BPW,), jnp.int32),
        pltpu.VMEM((_RCH, _D), jnp.float32),
        pltpu.VMEM((_RCH, _D), jnp.float32),
        pltpu.SemaphoreType.DMA,
        pltpu.SemaphoreType.DMA,
        pltpu.SemaphoreType.DMA,
        pltpu.SemaphoreType.DMA,
        pltpu.VMEM((_BPW * _NL,), jnp.int32),
        pltpu.VMEM((_BPW * _NL,), jnp.int32),
        pltpu.VMEM((_BPW * _NL,), jnp.float32),
        pltpu.VMEM((_BPW * _NL,), jnp.float32),
        pltpu.VMEM((_BPW + 16,), jnp.int32),
        pltpu.VMEM((_BPW + 16,), jnp.int32),
        pltpu.VMEM((_NCH, _RCH), jnp.int32),
    ],
)
def _sc_resolve(sk_hbm, qk_hbm, me_hbm, se_hbm, ml_hbm, sl_hbm,
                emb_hbm, log_hbm,
                map_v, qk_v, sk_v, ov_v, row_a, row_b,
                sem_ga, sem_gb, sem_wa, sem_wb,
                mli_v, sli_v, mlog_v, slog_v, sic_v, gif_v, gic_v):
    rows = (row_a, row_b)
    gsems = (sem_ga, sem_gb)
    wsems = (sem_wa, sem_wb)
    wid = lax.axis_index("s") * 2 + lax.axis_index("c")
    base = wid * _BPW

    pltpu.sync_copy(qk_hbm.at[pl.ds(base, _BPW)], qk_v)

    lanes = lax.broadcasted_iota(jnp.int32, (16,), 0)
    nxt = jnp.minimum(lanes + 1, 15)
    zeros16 = jnp.zeros((16,), jnp.int32)
    ones_mask = lanes >= 0

    def _init(i, carry):
        q = qk_v[pl.ds(i * 16, 16)]
        plsc.store_scatter(map_v, [q], zeros16, mask=ones_mask)
        return carry
    lax.fori_loop(0, _BPW // 16, _init, 0)

    def _start_gather(c):
        return pltpu.async_copy(me_hbm.at[qk_v.at[pl.ds(c * _RCH, _RCH)]],
                                rows[c % 2], gsems[c % 2])

    def _start_write(c):
        return pltpu.async_copy(rows[c % 2],
                                emb_hbm.at[pl.ds(base + c * _RCH, _RCH)],
                                wsems[c % 2])

    pend_g = {0: _start_gather(0)}
    pend_w = {}
    for c in range(_B // _SKCH):
        pltpu.sync_copy(sk_hbm.at[pl.ds(c * _SKCH, _SKCH)], sk_v)

        def _scan(i, carry, c=c):
            for u in range(8):
                v = i * 8 + u
                k = sk_v[pl.ds(v * 16, 16)]
                j = (c * _SKCH + v * 16) + lanes
                comb = jnp.sort((k << 14) | j)
                ks = comb >> 14
                kn = lax.gather(
                    ks, nxt[:, None],
                    lax.GatherDimensionNumbers(offset_dims=(),
                                               collapsed_slice_dims=(0,),
                                               start_index_map=(0,)),
                    slice_sizes=(1,),
                    mode=lax.GatherScatterMode.PROMISE_IN_BOUNDS)
                win = (ks != kn) | (lanes == 15)
                plsc.store_scatter(map_v, [ks], (comb & 16383) + 1, mask=win)
            return carry
        lax.fori_loop(0, _SKCH // 128, _scan, 0)

        if c < _NCH:
            pend_g.pop(c).wait()
            pend_w[c] = _start_write(c)
            if c + 1 < _NCH:
                if c - 1 in pend_w:
                    pend_w.pop(c - 1).wait()
                pend_g[c + 1] = _start_gather(c + 1)
    for c in sorted(pend_w):
        pend_w.pop(c).wait()

    def _resolve(i, carry):
        q = qk_v[pl.ds(i * 16, 16)]
        o = plsc.load_gather(map_v, [q])
        si = jnp.where(o > 0, o - 1, base + i * 16 + lanes)
        ov_v[pl.ds(i * 16, 16)] = o
        for t in range(_NL):
            mli_v[pl.ds(t * _BPW + i * 16, 16)] = t * _M + q
            sli_v[pl.ds(t * _BPW + i * 16, 16)] = t * _B + si
        return carry
    lax.fori_loop(0, _BPW // 16, _resolve, 0)

    def _prefill(i, carry):
        sic_v[pl.ds(i * 16, 16)] = base + i * 16 + lanes
        gif_v[pl.ds(i * 16, 16)] = _B + ((base + i * 16 + lanes) % _TRASH)
        return carry
    lax.fori_loop(0, _BPW // 16, _prefill, 0)

    def _compact(i, cnt):
        o = ov_v[pl.ds(i * 16, 16)]
        m = o > 0
        gi = base + i * 16 + lanes
        plsc.store_compressed(sic_v.at[pl.ds(cnt, 16)], o - 1, mask=m)
        plsc.store_compressed(gif_v.at[pl.ds(cnt, 16)], gi, mask=m)
        return cnt + jnp.sum(m.astype(jnp.int32))
    cnt = lax.fori_loop(0, _BPW // 16, _compact, 0)

    for c in range(_NCH):
        for k in range(_RCH // 16):
            gic_v[c, pl.ds(k * 16, 16)] = gif_v[pl.ds(c * _RCH + k * 16, 16)]

    ml_cp = pltpu.async_copy(ml_hbm.at[mli_v], mlog_v, sem_ga)
    sl_cp = pltpu.async_copy(sl_hbm.at[sli_v], slog_v, sem_gb)

    for c in range(_NCH):
        @pl.when(cnt > c * _RCH)
        def _do(c=c):
            pltpu.sync_copy(se_hbm.at[sic_v.at[pl.ds(c * _RCH, _RCH)]],
                            rows[c % 2])
            pltpu.sync_copy(rows[c % 2], emb_hbm.at[gic_v.at[c]])

    ml_cp.wait()
    sl_cp.wait()

    def _blend(i, carry):
        m = ov_v[pl.ds(i * 16, 16)] > 0
        for t in range(_NL):
            sl = pl.ds(t * _BPW + i * 16, 16)
            mlog_v[sl] = jnp.where(m, slog_v[sl], mlog_v[sl])
        return carry
    lax.fori_loop(0, _BPW // 16, _blend, 0)
    for t in range(_NL):
        pltpu.sync_copy(mlog_v.at[pl.ds(t * _BPW, _BPW)],
                        log_hbm.at[pl.ds(t * _B + base, _BPW)])


_RB = 2048


def _tc_body(emb_ref, lg_ref, w2_ref, sel_ref, o_ref):
    f = lax.dot_general(w2_ref[...], emb_ref[...], (((1,), (1,)), ((), ())),
                        preferred_element_type=jnp.float32)
    o_ref[...] = f + lax.dot_general(sel_ref[...], lg_ref[...],
                                     (((1,), (0,)), ((), ())),
                                     preferred_element_type=jnp.float32)


def _tc_combine(emb, lg, w2, sel):
    n_out = w2.shape[0]
    return pl.pallas_call(
        _tc_body,
        grid=(_B // _RB,),
        in_specs=[
            pl.BlockSpec((_RB, _D), lambda i: (i, 0)),
            pl.BlockSpec((_NL, _RB), lambda i: (0, i)),
            pl.BlockSpec((n_out, _D), lambda i: (0, 0)),
            pl.BlockSpec((n_out, _NL), lambda i: (0, 0)),
        ],
        out_specs=pl.BlockSpec((n_out, _RB), lambda i: (0, i)),
        out_shape=jax.ShapeDtypeStruct((n_out, _B), jnp.float32),
    )(emb, lg, w2, sel)


def kernel(mem_emb, mem_logits, store_keys, store_emb, store_logits,
           query_keys, W):
    sk = store_keys.astype(jnp.int32)
    qk = query_keys.astype(jnp.int32)
    emb, log_flat = _sc_resolve(
        sk, qk, mem_emb, store_emb,
        mem_logits.T.reshape(-1), store_logits.T.reshape(-1))
    lg = log_flat.reshape(_NL, _B)
    sd = W.shape[0]
    n_out = sd + _NL
    w2 = jnp.concatenate([W, jnp.zeros((_NL, _D), W.dtype)], axis=0)
    sel = jnp.zeros((n_out, _NL), jnp.float32).at[
        sd + jnp.arange(_NL), jnp.arange(_NL)].set(1.0)
    return _tc_combine(emb, lg, w2, sel).T


# speedup vs baseline: 1.0207x; 1.0207x over previous
import functools

import jax
import jax.numpy as jnp
from jax import lax
from jax.experimental import pallas as pl
from jax.experimental.pallas import tpu as pltpu
from jax.experimental.pallas import tpu_sc as plsc

_M = 100000
_MPAD = 100352
_D = 128
_NL = 4
_B = 16384
_NW = 32
_BPW = _B // _NW
_SKCH = 2048
_RCH = 64
_NCH = _BPW // _RCH
_TRASH = 2048

_mesh = plsc.VectorSubcoreMesh(core_axis_name="c", subcore_axis_name="s")


@functools.partial(
    pl.kernel,
    mesh=_mesh,
    compiler_params=pltpu.CompilerParams(needs_layout_passes=False),
    out_type=[
        jax.ShapeDtypeStruct((_B + _TRASH, _D), jnp.float32),
        jax.ShapeDtypeStruct((_B * _NL,), jnp.float32),
    ],
    scratch_types=[
        pltpu.VMEM((_MPAD,), jnp.int32),
        pltpu.VMEM((_BPW,), jnp.int32),
        pltpu.VMEM((_SKCH,), jnp.int32),
        pltpu.VMEM((_BPW,), jnp.int32),
        pltpu.VMEM((_RCH, _D), jnp.float32),
        pltpu.VMEM((_RCH, _D), jnp.float32),
        pltpu.SemaphoreType.DMA,
        pltpu.SemaphoreType.DMA,
        pltpu.SemaphoreType.DMA,
        pltpu.SemaphoreType.DMA,
        pltpu.VMEM((_BPW * _NL,), jnp.int32),
        pltpu.VMEM((_BPW * _NL,), jnp.int32),
        pltpu.VMEM((_BPW * _NL,), jnp.float32),
        pltpu.VMEM((_BPW * _NL,), jnp.float32),
        pltpu.VMEM((_BPW + 16,), jnp.int32),
        pltpu.VMEM((_BPW + 16,), jnp.int32),
        pltpu.VMEM((_NCH, _RCH), jnp.int32),
    ],
)
def _sc_resolve(sk_hbm, qk_hbm, me_hbm, se_hbm, ml_hbm, sl_hbm,
                emb_hbm, log_hbm,
                map_v, qk_v, sk_v, ov_v, row_a, row_b,
                sem_ga, sem_gb, sem_wa, sem_wb,
                mli_v, sli_v, mlog_v, slog_v, sic_v, gif_v, gic_v):
    rows = (row_a, row_b)
    gsems = (sem_ga, sem_gb)
    wsems = (sem_wa, sem_wb)
    wid = lax.axis_index("s") * 2 + lax.axis_index("c")
    base = wid * _BPW

    pltpu.sync_copy(qk_hbm.at[pl.ds(base, _BPW)], qk_v)

    lanes = lax.broadcasted_iota(jnp.int32, (16,), 0)
    nxt = jnp.minimum(lanes + 1, 15)
    zeros16 = jnp.zeros((16,), jnp.int32)
    ones_mask = lanes >= 0

    def _init(i, carry):
        q = qk_v[pl.ds(i * 16, 16)]
        plsc.store_scatter(map_v, [q], zeros16, mask=ones_mask)
        return carry
    lax.fori_loop(0, _BPW // 16, _init, 0)

    def _start_gather(c):
        return pltpu.async_copy(me_hbm.at[qk_v.at[pl.ds(c * _RCH, _RCH)]],
                                rows[c % 2], gsems[c % 2])

    def _start_write(c):
        return pltpu.async_copy(rows[c % 2],
                                emb_hbm.at[pl.ds(base + c * _RCH, _RCH)],
                                wsems[c % 2])

    pend_g = {0: _start_gather(0)}
    pend_w = {}
    for c in range(_B // _SKCH):
        pltpu.sync_copy(sk_hbm.at[pl.ds(c * _SKCH, _SKCH)], sk_v)

        def _scan(i, carry, c=c):
            for u in range(4):
                v = i * 4 + u
                k = sk_v[pl.ds(v * 16, 16)]
                j = (c * _SKCH + v * 16) + lanes
                comb = jnp.sort((k << 14) | j)
                ks = comb >> 14
                kn = lax.gather(
                    ks, nxt[:, None],
                    lax.GatherDimensionNumbers(offset_dims=(),
                                               collapsed_slice_dims=(0,),
                                               start_index_map=(0,)),
                    slice_sizes=(1,),
                    mode=lax.GatherScatterMode.PROMISE_IN_BOUNDS)
                win = (ks != kn) | (lanes == 15)
                plsc.store_scatter(map_v, [ks], (comb & 16383) + 1, mask=win)
            return carry
        lax.fori_loop(0, _SKCH // 64, _scan, 0)

        if c < _NCH:
            pend_g.pop(c).wait()
            pend_w[c] = _start_write(c)
            if c + 1 < _NCH:
                if c - 1 in pend_w:
                    pend_w.pop(c - 1).wait()
                pend_g[c + 1] = _start_gather(c + 1)
    for c in sorted(pend_w):
        pend_w.pop(c).wait()

    def _resolve(i, carry):
        q = qk_v[pl.ds(i * 16, 16)]
        o = plsc.load_gather(map_v, [q])
        si = jnp.where(o > 0, o - 1, base + i * 16 + lanes)
        ov_v[pl.ds(i * 16, 16)] = o
        pos = i * 16 + lanes
        for t in range(_NL):
            plsc.store_scatter(mli_v, [pos + t * _BPW], t * _M + q,
                               mask=ones_mask)
            plsc.store_scatter(sli_v, [pos + t * _BPW], t * _B + si,
                               mask=ones_mask)
        return carry
    lax.fori_loop(0, _BPW // 16, _resolve, 0)

    def _prefill(i, carry):
        sic_v[pl.ds(i * 16, 16)] = base + i * 16 + lanes
        gif_v[pl.ds(i * 16, 16)] = _B + ((base + i * 16 + lanes) % _TRASH)
        return carry
    lax.fori_loop(0, _BPW // 16, _prefill, 0)

    def _compact(i, cnt):
        o = ov_v[pl.ds(i * 16, 16)]
        m = o > 0
        gi = base + i * 16 + lanes
        plsc.store_compressed(sic_v.at[pl.ds(cnt, 16)], o - 1, mask=m)
        plsc.store_compressed(gif_v.at[pl.ds(cnt, 16)], gi, mask=m)
        return cnt + jnp.sum(m.astype(jnp.int32))
    cnt = lax.fori_loop(0, _BPW // 16, _compact, 0)

    for c in range(_NCH):
        for k in range(_RCH // 16):
            gic_v[c, pl.ds(k * 16, 16)] = gif_v[pl.ds(c * _RCH + k * 16, 16)]

    ml_cp = pltpu.async_copy(ml_hbm.at[mli_v], mlog_v, sem_ga)
    sl_cp = pltpu.async_copy(sl_hbm.at[sli_v], slog_v, sem_gb)

    for c in range(_NCH):
        @pl.when(cnt > c * _RCH)
        def _do(c=c):
            pltpu.sync_copy(se_hbm.at[sic_v.at[pl.ds(c * _RCH, _RCH)]],
                            rows[c % 2])
            pltpu.sync_copy(rows[c % 2], emb_hbm.at[gic_v.at[c]])

    ml_cp.wait()
    sl_cp.wait()

    def _blend(i, carry):
        m = ov_v[pl.ds(i * 16, 16)] > 0
        for t in range(_NL):
            sl = pl.ds(t * _BPW + i * 16, 16)
            mlog_v[sl] = jnp.where(m, slog_v[sl], mlog_v[sl])
        return carry
    lax.fori_loop(0, _BPW // 16, _blend, 0)
    for t in range(_NL):
        pltpu.sync_copy(mlog_v.at[pl.ds(t * _BPW, _BPW)],
                        log_hbm.at[pl.ds(t * _B + base, _BPW)])


_RB = 2048


def _tc_body(emb_ref, lg_ref, w2_ref, sel_ref, o_ref):
    f = lax.dot_general(w2_ref[...], emb_ref[...], (((1,), (1,)), ((), ())),
                        preferred_element_type=jnp.float32)
    o_ref[...] = f + lax.dot_general(sel_ref[...], lg_ref[...],
                                     (((1,), (0,)), ((), ())),
                                     preferred_element_type=jnp.float32)


def _tc_combine(emb, lg, w2, sel):
    n_out = w2.shape[0]
    return pl.pallas_call(
        _tc_body,
        grid=(_B // _RB,),
        in_specs=[
            pl.BlockSpec((_RB, _D), lambda i: (i, 0)),
            pl.BlockSpec((_NL, _RB), lambda i: (0, i)),
            pl.BlockSpec((n_out, _D), lambda i: (0, 0)),
            pl.BlockSpec((n_out, _NL), lambda i: (0, 0)),
        ],
        out_specs=pl.BlockSpec((n_out, _RB), lambda i: (0, i)),
        out_shape=jax.ShapeDtypeStruct((n_out, _B), jnp.float32),
    )(emb, lg, w2, sel)


def kernel(mem_emb, mem_logits, store_keys, store_emb, store_logits,
           query_keys, W):
    sk = store_keys.astype(jnp.int32)
    qk = query_keys.astype(jnp.int32)
    emb, log_flat = _sc_resolve(
        sk, qk, mem_emb, store_emb,
        mem_logits.T.reshape(-1), store_logits.T.reshape(-1))
    lg = log_flat.reshape(_NL, _B)
    sd = W.shape[0]
    n_out = sd + _NL
    w2 = jnp.concatenate([W, jnp.zeros((_NL, _D), W.dtype)], axis=0)
    sel = jnp.zeros((n_out, _NL), jnp.float32).at[
        sd + jnp.arange(_NL), jnp.arange(_NL)].set(1.0)
    return _tc_combine(emb, lg, w2, sel).T
